# baseline (speedup 1.0000x reference)
import functools
import numpy as np
import jax
import jax.numpy as jnp
from jax import lax
from jax.experimental import pallas as pl
from jax.experimental.pallas import tpu as pltpu
from jax.experimental.pallas import tpu_sc as plsc

_ALPHA = 4


def kernel(frames):
    B, T, C, H, W = frames.shape
    nsel = T // _ALPHA
    idx = [int(v) for v in np.linspace(0.0, T - 1, nsel).astype(np.int32)]

    info = plsc.get_sparse_core_info()
    NW = info.num_cores * info.num_subcores
    units = B * nsel * C
    per_w = units // NW

    mesh = plsc.VectorSubcoreMesh(core_axis_name="c", subcore_axis_name="s")

    @functools.partial(
        pl.kernel,
        mesh=mesh,
        out_type=jax.ShapeDtypeStruct((B, nsel, C, H, W), frames.dtype),
        scratch_types=[
            pltpu.VMEM((H, W), frames.dtype),
            pltpu.SemaphoreType.DMA,
        ],
    )
    def gather_k(frames_hbm, slow_hbm, buf, sem):
        wid = lax.axis_index("s") * info.num_cores + lax.axis_index("c")
        for i in range(per_w):
            u = wid * per_w + i
            c = u % C
            s = (u // C) % nsel
            b = u // (C * nsel)
            f = jnp.int32(0)
            for j, v in enumerate(idx):
                f = f + jnp.where(s == j, v, 0)
            pltpu.async_copy(frames_hbm.at[b, f, c], buf, sem).wait()
            pltpu.async_copy(buf, slow_hbm.at[b, s, c], sem).wait()

    slow = gather_k(frames)
    return (slow, frames)


# speedup vs baseline: 1.0454x; 1.0454x over previous
import functools
import numpy as np
import jax
import jax.numpy as jnp
from jax import lax
from jax.experimental import pallas as pl
from jax.experimental.pallas import tpu as pltpu
from jax.experimental.pallas import tpu_sc as plsc

_ALPHA = 4


def kernel(frames):
    B, T, C, H, W = frames.shape
    nsel = T // _ALPHA
    idx = [int(v) for v in np.linspace(0.0, T - 1, nsel).astype(np.int32)]

    info = plsc.get_sparse_core_info()
    NW = info.num_cores * info.num_subcores
    units = B * nsel * C
    per_w = units // NW

    mesh = plsc.VectorSubcoreMesh(core_axis_name="c", subcore_axis_name="s")

    @functools.partial(
        pl.kernel,
        mesh=mesh,
        out_type=jax.ShapeDtypeStruct((B, nsel, C, H, W), frames.dtype),
        scratch_types=[
            pltpu.VMEM((H, W), frames.dtype),
            pltpu.SemaphoreType.DMA,
        ],
    )
    def gather_k(frames_hbm, slow_hbm, buf, sem):
        wid = lax.axis_index("s") * info.num_cores + lax.axis_index("c")
        for i in range(per_w):
            u = wid * per_w + i
            c = u % C
            s = (u // C) % nsel
            b = u // (C * nsel)
            f = jnp.int32(0)
            for j, v in enumerate(idx):
                f = f + jnp.where(s == j, v, 0)
            pltpu.async_copy(frames_hbm.at[b, f, c], buf, sem).wait()
            pltpu.async_copy(buf, slow_hbm.at[b, s, c], sem).wait()

    slow = gather_k(frames)

    def copy_body(x_ref, fast_ref):
        fast_ref[...] = x_ref[...]

    blk = (B, 1, C, H, W)
    fast = pl.pallas_call(
        copy_body,
        grid=(T,),
        in_specs=[pl.BlockSpec(blk, lambda f: (0, f, 0, 0, 0))],
        out_specs=pl.BlockSpec(blk, lambda f: (0, f, 0, 0, 0)),
        out_shape=jax.ShapeDtypeStruct((B, T, C, H, W), frames.dtype),
    )(frames)
    return (slow, fast)


# speedup vs baseline: 1.3292x; 1.2715x over previous
import numpy as np
import jax
import jax.numpy as jnp
from jax.experimental import pallas as pl

_ALPHA = 4


def kernel(frames):
    B, T, C, H, W = frames.shape
    nsel = T // _ALPHA
    idx = [int(v) for v in np.linspace(0.0, T - 1, nsel).astype(np.int32)]
    TB = 2
    ngrp = T // TB

    def slot_of(g):
        s = 0
        for v in idx:
            s = s + jnp.where(g * TB > v, 1, 0)
        return s

    pos_in_grp = [0] * ngrp
    for v in idx:
        pos_in_grp[v // TB] = v % TB

    def body(x_ref, slow_ref, fast_ref):
        g = pl.program_id(0)
        v = x_ref[...]
        fast_ref[...] = v
        pos = 0
        for gi in range(ngrp):
            pos = pos + jnp.where(g == gi, pos_in_grp[gi], 0)
        slow_ref[...] = jnp.where(pos == 0, v[:, 0:1], v[:, 1:2])

    slow, fast = pl.pallas_call(
        body,
        grid=(ngrp,),
        in_specs=[pl.BlockSpec((B, TB, C, H, W), lambda g: (0, g, 0, 0, 0))],
        out_specs=[
            pl.BlockSpec((B, 1, C, H, W), lambda g: (0, slot_of(g), 0, 0, 0)),
            pl.BlockSpec((B, TB, C, H, W), lambda g: (0, g, 0, 0, 0)),
        ],
        out_shape=[
            jax.ShapeDtypeStruct((B, nsel, C, H, W), frames.dtype),
            jax.ShapeDtypeStruct((B, T, C, H, W), frames.dtype),
        ],
    )(frames)
    return (slow, fast)
